# Initial kernel scaffold; baseline (speedup 1.0000x reference)
import jax
import jax.numpy as jnp
from jax.experimental import pallas as pl


def kernel(features, predictions, fea_bank, score_bank, trg_idx):
    raise NotImplementedError("write your pallas kernel here")


# speedup vs baseline: 33.5482x; 33.5482x over previous
import functools

import jax
import jax.numpy as jnp
from jax.experimental import pallas as pl
from jax.experimental.pallas import tpu as pltpu

_NCAND = 6


def _prep_scatter_body(t_ref, feat_ref, pred_ref, fb_in, scb_in,
                       fb_ref, scb_ref, fn_ref, so_ref):
    del t_ref, fb_in, scb_in
    f = feat_ref[...]
    nrm = jnp.sqrt(jnp.sum(f * f, axis=2, keepdims=True))
    fn = f / jnp.maximum(nrm, 1e-12)
    fn_ref[...] = fn
    fb_ref[...] = fn
    p = pred_ref[...]
    e = jnp.exp(p - jnp.max(p, axis=2, keepdims=True))
    s = e / jnp.sum(e, axis=2, keepdims=True)
    so_ref[...] = s
    scb_ref[...] = s


def _hop_body(q_ref, b_ref, vals_ref, idx_ref, *, tn):
    s = jax.lax.dot_general(
        q_ref[...], b_ref[...], (((1,), (1,)), ((), ())),
        preferred_element_type=jnp.float32,
        precision=jax.lax.Precision.HIGHEST)
    base = pl.program_id(1) * tn
    cols = jax.lax.broadcasted_iota(jnp.int32, s.shape, 1)
    vs, ix = [], []
    cur = s
    for _ in range(_NCAND):
        a = jnp.argmax(cur, axis=1)
        vs.append(jnp.max(cur, axis=1))
        ix.append(base + a)
        cur = jnp.where(cols == a[:, None], -jnp.inf, cur)
    vals_ref[0, 0] = jnp.stack(vs, axis=1)
    idx_ref[0, 0] = jnp.stack(ix, axis=1)


def _merge_body(v_ref, i_ref, out_ref):
    v = v_ref[...]
    ii = i_ref[...]
    cols = jax.lax.broadcasted_iota(jnp.int32, v.shape, 1)
    outs = []
    cur = v
    for _ in range(_NCAND):
        a = jnp.argmax(cur, axis=1)
        oh = cols == a[:, None]
        outs.append(jnp.sum(jnp.where(oh, ii, 0), axis=1))
        cur = jnp.where(oh, -jnp.inf, cur)
    out_ref[...] = jnp.stack(outs, axis=1)


def _run_hop(q, bank, tq, tn):
    qn, d = q.shape
    n = bank.shape[0]
    nq, nt = qn // tq, n // tn
    vals, idxs = pl.pallas_call(
        functools.partial(_hop_body, tn=tn),
        grid=(nq, nt),
        in_specs=[pl.BlockSpec((tq, d), lambda iq, it: (iq, 0)),
                  pl.BlockSpec((tn, d), lambda iq, it: (it, 0))],
        out_specs=[pl.BlockSpec((1, 1, tq, _NCAND),
                                lambda iq, it: (iq, it, 0, 0)),
                   pl.BlockSpec((1, 1, tq, _NCAND),
                                lambda iq, it: (iq, it, 0, 0))],
        out_shape=[jax.ShapeDtypeStruct((nq, nt, tq, _NCAND), jnp.float32),
                   jax.ShapeDtypeStruct((nq, nt, tq, _NCAND), jnp.int32)],
    )(q, bank)
    nc = nt * _NCAND
    v2 = vals.transpose(0, 2, 1, 3).reshape(qn, nc)
    i2 = idxs.transpose(0, 2, 1, 3).reshape(qn, nc)
    return pl.pallas_call(
        _merge_body,
        grid=(nq,),
        in_specs=[pl.BlockSpec((tq, nc), lambda iq: (iq, 0)),
                  pl.BlockSpec((tq, nc), lambda iq: (iq, 0))],
        out_specs=pl.BlockSpec((tq, _NCAND), lambda iq: (iq, 0)),
        out_shape=jax.ShapeDtypeStruct((qn, _NCAND), jnp.int32),
    )(v2, i2)


def _gather_body(t_ref, *refs):
    del t_ref
    srcs, out_ref = refs[:-1], refs[-1]
    for j, s in enumerate(srcs):
        out_ref[0, j, :] = s[0, 0, :]


def _run_gather(src3, idx, g):
    n, _, w = src3.shape
    m = idx.shape[0]
    steps = m // g
    in_specs = [
        pl.BlockSpec((1, 1, w), (lambda i, t, j=j: (t[i * g + j], 0, 0)))
        for j in range(g)
    ]
    out = pl.pallas_call(
        _gather_body,
        grid_spec=pltpu.PrefetchScalarGridSpec(
            num_scalar_prefetch=1,
            grid=(steps,),
            in_specs=in_specs,
            out_specs=pl.BlockSpec((1, g, w), lambda i, t: (i, 0, 0)),
        ),
        out_shape=jax.ShapeDtypeStruct((steps, g, w), jnp.float32),
    )(idx, *([src3] * g))
    return out.reshape(m, w)


def _loss_body(so_ref, sn_ref, snkk_ref, idxnn_ref, trg_ref, out_ref):
    so = so_ref[...]
    sn = sn_ref[...]
    snkk = snkk_ref[...]
    so3 = so[:, None, :]
    kl2 = jnp.sum(sn * (jnp.log(sn) - so3), axis=2)
    eq = (idxnn_ref[...] == trg_ref[...]).astype(jnp.float32)
    match = jnp.sum(eq, axis=2)
    w = jnp.where(match > 0.0, match, 0.1)
    kl1 = jnp.sum(snkk * (jnp.log(snkk) - so3), axis=2)
    b = so.shape[0]
    ms = jnp.mean(so, axis=0, keepdims=True)
    gent = jnp.sum(ms * jnp.log(ms + 1e-5), axis=1, keepdims=True)
    t1 = jnp.sum(kl1, axis=(0, 1), keepdims=True)
    t2 = jnp.sum(kl2 * w, axis=(0, 1), keepdims=True)
    out_ref[...] = (t1 * 0.1 + t2) / b + gent


def kernel(features, predictions, fea_bank, score_bank, trg_idx):
    b, d = features.shape
    c = predictions.shape[1]
    n = fea_bank.shape[0]
    k = _NCAND - 1
    trg_idx = trg_idx.astype(jnp.int32)

    fb3, scb3, fn3, so3 = pl.pallas_call(
        _prep_scatter_body,
        grid_spec=pltpu.PrefetchScalarGridSpec(
            num_scalar_prefetch=1,
            grid=(b,),
            in_specs=[
                pl.BlockSpec((1, 1, d), lambda j, t: (j, 0, 0)),
                pl.BlockSpec((1, 1, c), lambda j, t: (j, 0, 0)),
                pl.BlockSpec(memory_space=pl.ANY),
                pl.BlockSpec(memory_space=pl.ANY),
            ],
            out_specs=[
                pl.BlockSpec((1, 1, d), lambda j, t: (t[j], 0, 0)),
                pl.BlockSpec((1, 1, c), lambda j, t: (t[j], 0, 0)),
                pl.BlockSpec((1, 1, d), lambda j, t: (j, 0, 0)),
                pl.BlockSpec((1, 1, c), lambda j, t: (j, 0, 0)),
            ],
        ),
        out_shape=[
            jax.ShapeDtypeStruct((n, 1, d), jnp.float32),
            jax.ShapeDtypeStruct((n, 1, c), jnp.float32),
            jax.ShapeDtypeStruct((b, 1, d), jnp.float32),
            jax.ShapeDtypeStruct((b, 1, c), jnp.float32),
        ],
        input_output_aliases={3: 0, 4: 1},
    )(trg_idx, features.reshape(b, 1, d), predictions.reshape(b, 1, c),
      fea_bank.reshape(n, 1, d), score_bank.reshape(n, 1, c))

    fb = fb3.reshape(n, d)
    fn = fn3.reshape(b, d)
    so = so3.reshape(b, c)

    top1 = _run_hop(fn, fb, b, 5000)
    idx_near = top1[:, 1:].reshape(b * k)

    q2 = _run_gather(fb3, idx_near, 8)
    top2 = _run_hop(q2, fb, b, 5000)
    idx_nn = top2[:, 1:]

    all_idx = jnp.concatenate([idx_near, idx_nn.reshape(b * k * k)])
    g = _run_gather(scb3, all_idx, 16)
    sn3 = g[:b * k].reshape(b, k, c)
    snkk3 = g[b * k:].reshape(b, k * k, c)

    out = pl.pallas_call(
        _loss_body,
        out_shape=jax.ShapeDtypeStruct((1, 1), jnp.float32),
    )(so, sn3, snkk3, idx_nn.reshape(b, k, k), trg_idx.reshape(b, 1, 1))
    return out.reshape(())


# speedup vs baseline: 37.7217x; 1.1244x over previous
import functools

import jax
import jax.numpy as jnp
from jax.experimental import pallas as pl
from jax.experimental.pallas import tpu as pltpu

_NCAND = 6


def _prep_body(f_ref, p_ref, fn_ref, so_ref):
    f = f_ref[...]
    nrm = jnp.sqrt(jnp.sum(f * f, axis=1, keepdims=True))
    fn_ref[...] = f / jnp.maximum(nrm, 1e-12)
    p = p_ref[...]
    e = jnp.exp(p - jnp.max(p, axis=1, keepdims=True))
    so_ref[...] = e / jnp.sum(e, axis=1, keepdims=True)


def _scatter_body(t_ref, fn_ref, so_ref, fb_in, scb_in, fb_ref, scb_ref):
    del t_ref, fb_in, scb_in
    fb_ref[...] = fn_ref[...]
    scb_ref[...] = so_ref[...]


def _hop_body(q_ref, b_ref, vals_ref, idx_ref, *, tn):
    tq = q_ref.shape[0]
    s = jax.lax.dot_general(
        q_ref[...], b_ref[...], (((1,), (1,)), ((), ())),
        preferred_element_type=jnp.float32,
        precision=jax.lax.Precision.HIGHEST)
    base = pl.program_id(1) * tn
    cols = jax.lax.broadcasted_iota(jnp.int32, (tq, tn), 1)
    vl, il = [], []
    for _ in range(_NCAND):
        m = jnp.max(s, axis=1, keepdims=True)
        gi = jnp.min(jnp.where(s == m, cols, 2**31 - 1),
                     axis=1, keepdims=True)
        vl.append(m[:, 0])
        il.append(base + gi[:, 0])
        s = jnp.where(cols == gi, -jnp.inf, s)
    vals_ref[0, 0] = jnp.stack(vl, axis=1)
    idx_ref[0, 0] = jnp.stack(il, axis=1)


def _merge_body(v_ref, i_ref, out_ref, *, n_real):
    v = v_ref[...]
    ii = i_ref[...]
    v = jnp.where(ii < n_real, v, -jnp.inf)
    outs = []
    for _ in range(_NCAND):
        m = jnp.max(v, axis=1, keepdims=True)
        big = jnp.full_like(ii, 2**31 - 1)
        gi = jnp.min(jnp.where(v == m, ii, big), axis=1, keepdims=True)
        outs.append(gi)
        v = jnp.where(ii == gi, -jnp.inf, v)
    out_ref[...] = jnp.concatenate(outs, axis=1)


def _run_hop(q, bank, tq, tn, n_real):
    qn, d = q.shape
    n = bank.shape[0]
    nq, nt = qn // tq, n // tn
    npt = _NCAND
    vals, idxs = pl.pallas_call(
        functools.partial(_hop_body, tn=tn),
        grid=(nq, nt),
        in_specs=[pl.BlockSpec((tq, d), lambda iq, it: (iq, 0)),
                  pl.BlockSpec((tn, d), lambda iq, it: (it, 0))],
        out_specs=[pl.BlockSpec((1, 1, tq, npt),
                                lambda iq, it: (iq, it, 0, 0)),
                   pl.BlockSpec((1, 1, tq, npt),
                                lambda iq, it: (iq, it, 0, 0))],
        out_shape=[jax.ShapeDtypeStruct((nq, nt, tq, npt), jnp.float32),
                   jax.ShapeDtypeStruct((nq, nt, tq, npt), jnp.int32)],
    )(q, bank)
    nc = nt * npt
    v2 = vals.transpose(0, 2, 1, 3).reshape(qn, nc)
    i2 = idxs.transpose(0, 2, 1, 3).reshape(qn, nc)
    return pl.pallas_call(
        functools.partial(_merge_body, n_real=n_real),
        grid=(nq,),
        in_specs=[pl.BlockSpec((tq, nc), lambda iq: (iq, 0)),
                  pl.BlockSpec((tq, nc), lambda iq: (iq, 0))],
        out_specs=pl.BlockSpec((tq, _NCAND), lambda iq: (iq, 0)),
        out_shape=jax.ShapeDtypeStruct((qn, _NCAND), jnp.int32),
    )(v2, i2)


def _gather_body(t_ref, *refs):
    del t_ref
    srcs, out_ref = refs[:-1], refs[-1]
    for j, s in enumerate(srcs):
        out_ref[0, j, :] = s[0, 0, :]


def _run_gather(src3, idx, g):
    n, _, w = src3.shape
    m = idx.shape[0]
    steps = m // g
    in_specs = [
        pl.BlockSpec((1, 1, w), (lambda i, t, j=j: (t[i * g + j], 0, 0)))
        for j in range(g)
    ]
    out = pl.pallas_call(
        _gather_body,
        grid_spec=pltpu.PrefetchScalarGridSpec(
            num_scalar_prefetch=1,
            grid=(steps,),
            in_specs=in_specs,
            out_specs=pl.BlockSpec((1, g, w), lambda i, t: (i, 0, 0)),
        ),
        out_shape=jax.ShapeDtypeStruct((steps, g, w), jnp.float32),
    )(idx, *([src3] * g))
    return out.reshape(m, w)


def _loss_body(so_ref, sn_ref, snkk_ref, idxnn_ref, trg_ref, out_ref):
    so = so_ref[...]
    sn = sn_ref[...]
    snkk = snkk_ref[...]
    so3 = so[:, None, :]
    kl2 = jnp.sum(sn * (jnp.log(sn) - so3), axis=2)
    eq = (idxnn_ref[...] == trg_ref[...]).astype(jnp.float32)
    match = jnp.sum(eq, axis=2)
    w = jnp.where(match > 0.0, match, 0.1)
    kl1 = jnp.sum(snkk * (jnp.log(snkk) - so3), axis=2)
    b = so.shape[0]
    ms = jnp.mean(so, axis=0, keepdims=True)
    gent = jnp.sum(ms * jnp.log(ms + 1e-5), axis=1, keepdims=True)
    t1 = jnp.sum(kl1, axis=(0, 1), keepdims=True)
    t2 = jnp.sum(kl2 * w, axis=(0, 1), keepdims=True)
    out_ref[...] = (t1 * 0.1 + t2) / b + gent


def kernel(features, predictions, fea_bank, score_bank, trg_idx):
    b, d = features.shape
    c = predictions.shape[1]
    n = fea_bank.shape[0]
    k = _NCAND - 1
    tn = 5120
    n_pad = ((n + tn - 1) // tn) * tn
    trg_idx = trg_idx.astype(jnp.int32)

    fn, so = pl.pallas_call(
        _prep_body,
        out_shape=[jax.ShapeDtypeStruct((b, d), jnp.float32),
                   jax.ShapeDtypeStruct((b, c), jnp.float32)],
    )(features, predictions)

    fb_padded = jnp.pad(fea_bank, ((0, n_pad - n), (0, 0)))
    fbp3, scb3 = pl.pallas_call(
        _scatter_body,
        grid_spec=pltpu.PrefetchScalarGridSpec(
            num_scalar_prefetch=1,
            grid=(b,),
            in_specs=[
                pl.BlockSpec((1, 1, d), lambda j, t: (j, 0, 0)),
                pl.BlockSpec((1, 1, c), lambda j, t: (j, 0, 0)),
                pl.BlockSpec(memory_space=pl.ANY),
                pl.BlockSpec(memory_space=pl.ANY),
            ],
            out_specs=[
                pl.BlockSpec((1, 1, d), lambda j, t: (t[j], 0, 0)),
                pl.BlockSpec((1, 1, c), lambda j, t: (t[j], 0, 0)),
            ],
        ),
        out_shape=[
            jax.ShapeDtypeStruct((n_pad, 1, d), jnp.float32),
            jax.ShapeDtypeStruct((n, 1, c), jnp.float32),
        ],
        input_output_aliases={3: 0, 4: 1},
    )(trg_idx, fn.reshape(b, 1, d), so.reshape(b, 1, c),
      fb_padded.reshape(n_pad, 1, d), score_bank.reshape(n, 1, c))

    fb = fbp3.reshape(n_pad, d)

    top1 = _run_hop(fn, fb, b, tn, n)
    idx_near = top1[:, 1:].reshape(b * k)

    q2 = _run_gather(fbp3, idx_near, 16)
    top2 = _run_hop(q2, fb, b, tn, n)
    idx_nn = top2[:, 1:]

    all_idx = jnp.concatenate([idx_near, idx_nn.reshape(b * k * k)])
    g = _run_gather(scb3, all_idx, 32)
    sn3 = g[:b * k].reshape(b, k, c)
    snkk3 = g[b * k:].reshape(b, k * k, c)

    out = pl.pallas_call(
        _loss_body,
        out_shape=jax.ShapeDtypeStruct((1, 1), jnp.float32),
    )(so, sn3, snkk3, idx_nn.reshape(b, k, k), trg_idx.reshape(b, 1, 1))
    return out.reshape(())


# speedup vs baseline: 45.6441x; 1.2100x over previous
import functools

import jax
import jax.numpy as jnp
from jax.experimental import pallas as pl
from jax.experimental.pallas import tpu as pltpu
from jax.experimental.pallas import tpu_sc as plsc

_NCAND = 6
_SC_CORES = 2
_SC_SUBCORES = 16


def _prep_body(f_ref, p_ref, fn_ref, so_ref):
    f = f_ref[...]
    nrm = jnp.sqrt(jnp.sum(f * f, axis=1, keepdims=True))
    fn_ref[...] = f / jnp.maximum(nrm, 1e-12)
    p = p_ref[...]
    e = jnp.exp(p - jnp.max(p, axis=1, keepdims=True))
    so_ref[...] = e / jnp.sum(e, axis=1, keepdims=True)


def _scatter_body(t_ref, fn_ref, so_ref, fb_in, scb_in, fb_ref, scb_ref):
    del t_ref, fb_in, scb_in
    fb_ref[...] = fn_ref[...]
    scb_ref[...] = so_ref[...]


def _hop_body(q_ref, b_ref, vals_ref, idx_ref, *, tn):
    tq = q_ref.shape[0]
    s = jax.lax.dot_general(
        q_ref[...], b_ref[...], (((1,), (1,)), ((), ())),
        preferred_element_type=jnp.float32,
        precision=jax.lax.Precision.HIGHEST)
    base = pl.program_id(1) * tn
    cols = jax.lax.broadcasted_iota(jnp.int32, (tq, tn), 1)
    vl, il = [], []
    for _ in range(_NCAND):
        m = jnp.max(s, axis=1, keepdims=True)
        gi = jnp.min(jnp.where(s == m, cols, 2**31 - 1),
                     axis=1, keepdims=True)
        vl.append(m[:, 0])
        il.append(base + gi[:, 0])
        s = jnp.where(cols == gi, -jnp.inf, s)
    vals_ref[0, 0] = jnp.stack(vl, axis=1)
    idx_ref[0, 0] = jnp.stack(il, axis=1)


def _merge_body(v_ref, i_ref, out_ref, *, n_real):
    v = v_ref[...]
    ii = i_ref[...]
    v = jnp.where(ii < n_real, v, -jnp.inf)
    outs = []
    for _ in range(_NCAND):
        m = jnp.max(v, axis=1, keepdims=True)
        big = jnp.full_like(ii, 2**31 - 1)
        gi = jnp.min(jnp.where(v == m, ii, big), axis=1, keepdims=True)
        outs.append(gi)
        v = jnp.where(ii == gi, -jnp.inf, v)
    out_ref[...] = jnp.concatenate(outs, axis=1)


def _run_hop(q, bank, tq, tn, n_real):
    qn, d = q.shape
    n = bank.shape[0]
    nq, nt = qn // tq, n // tn
    npt = _NCAND
    vals, idxs = pl.pallas_call(
        functools.partial(_hop_body, tn=tn),
        grid=(nq, nt),
        in_specs=[pl.BlockSpec((tq, d), lambda iq, it: (iq, 0)),
                  pl.BlockSpec((tn, d), lambda iq, it: (it, 0))],
        out_specs=[pl.BlockSpec((1, 1, tq, npt),
                                lambda iq, it: (iq, it, 0, 0)),
                   pl.BlockSpec((1, 1, tq, npt),
                                lambda iq, it: (iq, it, 0, 0))],
        out_shape=[jax.ShapeDtypeStruct((nq, nt, tq, npt), jnp.float32),
                   jax.ShapeDtypeStruct((nq, nt, tq, npt), jnp.int32)],
    )(q, bank)
    nc = nt * npt
    v2 = vals.transpose(0, 2, 1, 3).reshape(qn, nc)
    i2 = idxs.transpose(0, 2, 1, 3).reshape(qn, nc)
    return pl.pallas_call(
        functools.partial(_merge_body, n_real=n_real),
        grid=(nq,),
        in_specs=[pl.BlockSpec((tq, nc), lambda iq: (iq, 0)),
                  pl.BlockSpec((tq, nc), lambda iq: (iq, 0))],
        out_specs=pl.BlockSpec((tq, _NCAND), lambda iq: (iq, 0)),
        out_shape=jax.ShapeDtypeStruct((qn, _NCAND), jnp.int32),
    )(v2, i2)


def _sc_gather(table, idx):
    _, d = table.shape
    m = idx.shape[0]
    nw = _SC_CORES * _SC_SUBCORES
    b_per_w = m // nw
    mesh = plsc.VectorSubcoreMesh(core_axis_name="c", subcore_axis_name="s")

    def body(table_hbm, idx_hbm, out_hbm, idx_v, rows_v, sem):
        wid = jax.lax.axis_index("s") * _SC_CORES + jax.lax.axis_index("c")
        base = wid * b_per_w
        pltpu.sync_copy(idx_hbm.at[pl.ds(base, b_per_w)], idx_v)
        pltpu.async_copy(table_hbm.at[idx_v], rows_v, sem).wait()
        pltpu.sync_copy(rows_v, out_hbm.at[pl.ds(base, b_per_w)])

    return pl.kernel(
        body,
        out_type=jax.ShapeDtypeStruct((m, d), jnp.float32),
        mesh=mesh,
        scratch_types=[
            pltpu.VMEM((b_per_w,), jnp.int32),
            pltpu.VMEM((b_per_w, d), jnp.float32),
            pltpu.SemaphoreType.DMA,
        ],
    )(table, idx)


def _loss_body(so_ref, sn_ref, snkk_ref, idxnn_ref, trg_ref, out_ref):
    so = so_ref[...]
    sn = sn_ref[...]
    snkk = snkk_ref[...]
    so3 = so[:, None, :]
    kl2 = jnp.sum(sn * (jnp.log(sn) - so3), axis=2)
    eq = (idxnn_ref[...] == trg_ref[...]).astype(jnp.float32)
    match = jnp.sum(eq, axis=2)
    w = jnp.where(match > 0.0, match, 0.1)
    kl1 = jnp.sum(snkk * (jnp.log(snkk) - so3), axis=2)
    b = so.shape[0]
    ms = jnp.mean(so, axis=0, keepdims=True)
    gent = jnp.sum(ms * jnp.log(ms + 1e-5), axis=1, keepdims=True)
    t1 = jnp.sum(kl1, axis=(0, 1), keepdims=True)
    t2 = jnp.sum(kl2 * w, axis=(0, 1), keepdims=True)
    out_ref[...] = (t1 * 0.1 + t2) / b + gent


def kernel(features, predictions, fea_bank, score_bank, trg_idx):
    b, d = features.shape
    c = predictions.shape[1]
    n = fea_bank.shape[0]
    k = _NCAND - 1
    tn = 5120
    n_pad = ((n + tn - 1) // tn) * tn
    trg_idx = trg_idx.astype(jnp.int32)

    fn, so = pl.pallas_call(
        _prep_body,
        out_shape=[jax.ShapeDtypeStruct((b, d), jnp.float32),
                   jax.ShapeDtypeStruct((b, c), jnp.float32)],
    )(features, predictions)

    fb_padded = jnp.pad(fea_bank, ((0, n_pad - n), (0, 0)))
    scb_padded = jnp.pad(score_bank, ((0, 0), (0, d - c)))
    so_padded = jnp.pad(so, ((0, 0), (0, d - c)))
    fbp3, scbp3 = pl.pallas_call(
        _scatter_body,
        grid_spec=pltpu.PrefetchScalarGridSpec(
            num_scalar_prefetch=1,
            grid=(b,),
            in_specs=[
                pl.BlockSpec((1, 1, d), lambda j, t: (j, 0, 0)),
                pl.BlockSpec((1, 1, d), lambda j, t: (j, 0, 0)),
                pl.BlockSpec(memory_space=pl.ANY),
                pl.BlockSpec(memory_space=pl.ANY),
            ],
            out_specs=[
                pl.BlockSpec((1, 1, d), lambda j, t: (t[j], 0, 0)),
                pl.BlockSpec((1, 1, d), lambda j, t: (t[j], 0, 0)),
            ],
        ),
        out_shape=[
            jax.ShapeDtypeStruct((n_pad, 1, d), jnp.float32),
            jax.ShapeDtypeStruct((n, 1, d), jnp.float32),
        ],
        input_output_aliases={3: 0, 4: 1},
    )(trg_idx, fn.reshape(b, 1, d), so_padded.reshape(b, 1, d),
      fb_padded.reshape(n_pad, 1, d), scb_padded.reshape(n, 1, d))

    fb = fbp3.reshape(n_pad, d)
    scbp = scbp3.reshape(n, d)

    top1 = _run_hop(fn, fb, b, tn, n)
    idx_near = top1[:, 1:].reshape(b * k)

    q2 = _sc_gather(fb, idx_near)
    sn_rows = _sc_gather(scbp, idx_near)[:, :c]
    top2 = _run_hop(q2, fb, b, tn, n)
    idx_nn = top2[:, 1:]

    snkk_rows = _sc_gather(scbp, idx_nn.reshape(b * k * k))[:, :c]
    sn3 = sn_rows.reshape(b, k, c)
    snkk3 = snkk_rows.reshape(b, k * k, c)

    out = pl.pallas_call(
        _loss_body,
        out_shape=jax.ShapeDtypeStruct((1, 1), jnp.float32),
    )(so, sn3, snkk3, idx_nn.reshape(b, k, k), trg_idx.reshape(b, 1, 1))
    return out.reshape(())
